# baseline (speedup 1.0000x reference)
import jax
import jax.numpy as jnp
from jax.experimental import pallas as pl
from jax.experimental.pallas import tpu as pltpu

_NUM_ROWS = 64
_ROW_STRIDE = 700
_D = 256


def _tc_body(table_hbm, out_vmem, sem):
    copies = []
    for j in range(_NUM_ROWS):
        copies.append(
            pltpu.make_async_copy(
                table_hbm.at[pl.ds(j * _ROW_STRIDE, 1)],
                out_vmem.at[pl.ds(j, 1)],
                sem,
            )
        )
    for c in copies:
        c.start()
    pltpu.make_async_copy(
        table_hbm.at[pl.ds(0, _NUM_ROWS)], out_vmem, sem
    ).wait()


def kernel(inputs):
    gathered = pl.pallas_call(
        _tc_body,
        out_shape=jax.ShapeDtypeStruct((_NUM_ROWS, _D), jnp.float32),
        in_specs=[pl.BlockSpec(memory_space=pltpu.MemorySpace.HBM)],
        out_specs=pl.BlockSpec(memory_space=pltpu.MemorySpace.VMEM),
        scratch_shapes=[pltpu.SemaphoreType.DMA],
    )(inputs)
    return jnp.reshape(gathered, (1, _NUM_ROWS * _D))


# speedup vs baseline: 1.0142x; 1.0142x over previous
import jax
import jax.numpy as jnp
from jax.experimental import pallas as pl
from jax.experimental.pallas import tpu as pltpu

_NUM_ROWS = 64
_ROW_STRIDE = 700
_D = 256


def _tc_body(table_hbm, out_vmem, sem):
    copies = []
    for j in range(_NUM_ROWS):
        copies.append(
            pltpu.make_async_copy(
                table_hbm.at[pl.ds(j * _ROW_STRIDE, 1)],
                out_vmem.at[pl.ds(j, 1)],
                sem,
            )
        )
    for c in copies:
        c.start()
    for c in copies:
        c.wait()


def kernel(inputs):
    gathered = pl.pallas_call(
        _tc_body,
        out_shape=jax.ShapeDtypeStruct((_NUM_ROWS, _D), jnp.float32),
        in_specs=[pl.BlockSpec(memory_space=pltpu.MemorySpace.HBM)],
        out_specs=pl.BlockSpec(memory_space=pltpu.MemorySpace.VMEM),
        scratch_shapes=[pltpu.SemaphoreType.DMA],
    )(inputs)
    return jnp.reshape(gathered, (1, _NUM_ROWS * _D))
